# baseline (speedup 1.0000x reference)
import jax
import jax.numpy as jnp
from jax import lax
from jax.experimental import pallas as pl
from jax.experimental.pallas import tpu as pltpu
from jax.experimental.pallas import tpu_sc as plsc

B, T, N, C = 64, 12, 10000, 3
TIMES = 288
DAYS = 7
F = 64
NC, NS, L = 2, 16, 16
NW = NC * NS
B_PER_W = B // NW
FQ = 4
TOD_STRIDE = F + 1
DOW_STRIDE = DAYS * F + 1


def _body(x_hbm, tod_hbm, dow_hbm, out_hbm,
          slab_v, pidx_v, todp_v, dowr_v, fus_v, rows_v, sems):
    wid = lax.axis_index("s") * NC + lax.axis_index("c")
    iota = lax.broadcasted_iota(jnp.int32, (L,), 0)

    pltpu.sync_copy(tod_hbm, slab_v.at[pl.ds(0, TIMES * F)])

    @plsc.parallel_loop(0, TIMES, step=1, unroll=4)
    def _(k):
        for i in range(F // L):
            todp_v[pl.ds(k * TOD_STRIDE + L * i, L)] = (
                slab_v[pl.ds(k * F + L * i, L)])

    pltpu.sync_copy(dow_hbm, slab_v.at[pl.ds(0, DAYS * F)])

    @plsc.parallel_loop(0, L, step=1)
    def _(l):
        for i in range(DAYS * F // L):
            dowr_v[pl.ds(l * DOW_STRIDE + L * i, L)] = (
                slab_v[pl.ds(L * i, L)])

    lane_off = DOW_STRIDE * iota

    for rb in range(B_PER_W):
        b = wid * B_PER_W + rb
        pltpu.sync_copy(x_hbm.at[b, 0], slab_v)

        def idx_body(i, dminmax):
            n0 = i * L
            base = 3 * n0 + 3 * iota
            v1 = plsc.load_gather(slab_v, [base + 1])
            v2 = plsc.load_gather(slab_v, [base + 2])
            k65 = (v1 * TIMES).astype(jnp.int32) * TOD_STRIDE
            d = v2.astype(jnp.int32)
            pidx_v[pl.ds(n0, L)] = k65 | ((d * F) << 16)
            return (jnp.minimum(dminmax[0], jnp.min(d)),
                    jnp.maximum(dminmax[1], jnp.max(d)))

        dmin, dmax = lax.fori_loop(0, N // L, idx_body,
                                   (jnp.int32(DAYS), jnp.int32(-1)))

        uniform_d = dmin == dmax

        @pl.when(uniform_d)
        def _():
            d0 = dmin * F

            @plsc.parallel_loop(0, TIMES, step=1, unroll=4)
            def _(kk):
                for i in range(F // L):
                    fus_v[pl.ds(kk * TOD_STRIDE + L * i, L)] = (
                        todp_v[pl.ds(kk * TOD_STRIDE + L * i, L)]
                        + dowr_v[pl.ds(d0 + L * i, L)])

        def quad_body(cc, _):
            f0 = FQ * cc

            @pl.when(cc > 0)
            def _():
                for j in range(FQ):
                    pltpu.make_async_copy(
                        rows_v[j], out_hbm.at[b, 0], sems[j]).wait()

            @pl.when(uniform_d)
            def _():
                @plsc.parallel_loop(0, N, step=L, unroll=25)
                def _(n0):
                    k = pidx_v[pl.ds(n0, L)] & 0xFFFF
                    for j in range(FQ):
                        rows_v[j][pl.ds(n0, L)] = (
                            plsc.load_gather(fus_v, [k + (f0 + j)]))

            @pl.when(jnp.logical_not(uniform_d))
            def _():
                @plsc.parallel_loop(0, N, step=L, unroll=5)
                def _(n0):
                    p = pidx_v[pl.ds(n0, L)]
                    k = p & 0xFFFF
                    d = lane_off + (p >> 16)
                    for j in range(FQ):
                        rows_v[j][pl.ds(n0, L)] = (
                            plsc.load_gather(todp_v, [k + (f0 + j)])
                            + plsc.load_gather(dowr_v, [d + (f0 + j)]))

            for j in range(FQ):
                pltpu.async_copy(rows_v[j], out_hbm.at[b, f0 + j], sems[j])
            return 0

        lax.fori_loop(0, F // FQ, quad_body, 0)
        for j in range(FQ):
            pltpu.make_async_copy(rows_v[j], out_hbm.at[b, 0], sems[j]).wait()


def kernel(x, time_of_day, day_of_week):
    mesh = plsc.VectorSubcoreMesh(core_axis_name="c", subcore_axis_name="s",
                                  num_cores=NC, num_subcores=NS)
    out = pl.kernel(
        _body,
        out_type=jax.ShapeDtypeStruct((B, F, N), jnp.float32),
        mesh=mesh,
        compiler_params=pltpu.CompilerParams(needs_layout_passes=False),
        scratch_types=[
            pltpu.VMEM((N * C,), jnp.float32),
            pltpu.VMEM((N,), jnp.int32),
            pltpu.VMEM((TIMES * TOD_STRIDE,), jnp.float32),
            pltpu.VMEM((L * DOW_STRIDE,), jnp.float32),
            pltpu.VMEM((TIMES * TOD_STRIDE,), jnp.float32),
            [pltpu.VMEM((N,), jnp.float32)] * FQ,
            [pltpu.SemaphoreType.DMA] * FQ,
        ],
    )(x[:, -1].reshape(B, 1, N * C),
      time_of_day.reshape(-1), day_of_week.reshape(-1))
    return out[..., None]


# speedup vs baseline: 1.1549x; 1.1549x over previous
import jax
import jax.numpy as jnp
from jax import lax
from jax.experimental import pallas as pl
from jax.experimental.pallas import tpu as pltpu
from jax.experimental.pallas import tpu_sc as plsc

B, T, N, C = 64, 12, 10000, 3
TIMES = 288
DAYS = 7
F = 64
NC, NS, L = 2, 16, 16
NW = NC * NS
B_PER_W = B // NW
FQ = 4
TOD_STRIDE = F + 1
DOW_STRIDE = DAYS * F + 1


def _body(x_hbm, tod_hbm, dow_hbm, out_hbm,
          slab_v, pidx_v, todp_v, dowr_v, fus_v, bufa_v, bufb_v, sem_a, sem_b):
    wid = lax.axis_index("s") * NC + lax.axis_index("c")
    iota = lax.broadcasted_iota(jnp.int32, (L,), 0)

    pltpu.sync_copy(tod_hbm, slab_v.at[pl.ds(0, TIMES * F)])

    @plsc.parallel_loop(0, TIMES, step=1, unroll=4)
    def _(k):
        for i in range(F // L):
            todp_v[pl.ds(k * TOD_STRIDE + L * i, L)] = (
                slab_v[pl.ds(k * F + L * i, L)])

    pltpu.sync_copy(dow_hbm, slab_v.at[pl.ds(0, DAYS * F)])

    @plsc.parallel_loop(0, L, step=1)
    def _(l):
        for i in range(DAYS * F // L):
            dowr_v[pl.ds(l * DOW_STRIDE + L * i, L)] = (
                slab_v[pl.ds(L * i, L)])

    lane_off = DOW_STRIDE * iota

    for rb in range(B_PER_W):
        b = wid * B_PER_W + rb
        pltpu.sync_copy(x_hbm.at[b, 0], slab_v)

        def idx_body(i, dminmax):
            n0 = i * L
            base = 3 * n0 + 3 * iota
            v1 = plsc.load_gather(slab_v, [base + 1])
            v2 = plsc.load_gather(slab_v, [base + 2])
            k65 = (v1 * TIMES).astype(jnp.int32) * TOD_STRIDE
            d = v2.astype(jnp.int32)
            pidx_v[pl.ds(n0, L)] = k65 | ((d * F) << 16)
            return (jnp.minimum(dminmax[0], jnp.min(d)),
                    jnp.maximum(dminmax[1], jnp.max(d)))

        dmin, dmax = lax.fori_loop(0, N // L, idx_body,
                                   (jnp.int32(DAYS), jnp.int32(-1)))

        uniform_d = dmin == dmax

        @pl.when(uniform_d)
        def _():
            d0 = dmin * F

            @plsc.parallel_loop(0, TIMES, step=1, unroll=4)
            def _(kk):
                for i in range(F // L):
                    fus_v[pl.ds(kk * TOD_STRIDE + L * i, L)] = (
                        todp_v[pl.ds(kk * TOD_STRIDE + L * i, L)]
                        + dowr_v[pl.ds(d0 + L * i, L)])

        def compute_pair(f0, buf):
            @pl.when(uniform_d)
            def _():
                @plsc.parallel_loop(0, N, step=L, unroll=5)
                def _(n0):
                    k = pidx_v[pl.ds(n0, L)] & 0xFFFF
                    for j in range(2):
                        buf[j, pl.ds(n0, L)] = (
                            plsc.load_gather(fus_v, [k + (f0 + j)]))

            @pl.when(jnp.logical_not(uniform_d))
            def _():
                @plsc.parallel_loop(0, N, step=L, unroll=5)
                def _(n0):
                    p = pidx_v[pl.ds(n0, L)]
                    k = p & 0xFFFF
                    d = lane_off + (p >> 16)
                    for j in range(2):
                        buf[j, pl.ds(n0, L)] = (
                            plsc.load_gather(todp_v, [k + (f0 + j)])
                            + plsc.load_gather(dowr_v, [d + (f0 + j)]))

        def quad_body(cc, _):
            f0 = FQ * cc
            row0 = b * F + f0

            @pl.when(cc > 0)
            def _():
                pltpu.make_async_copy(
                    bufa_v, out_hbm.at[pl.ds(0, 2)], sem_a).wait()

            compute_pair(f0, bufa_v)
            pltpu.async_copy(bufa_v, out_hbm.at[pl.ds(row0, 2)], sem_a)

            @pl.when(cc > 0)
            def _():
                pltpu.make_async_copy(
                    bufb_v, out_hbm.at[pl.ds(0, 2)], sem_b).wait()

            compute_pair(f0 + 2, bufb_v)
            pltpu.async_copy(bufb_v, out_hbm.at[pl.ds(row0 + 2, 2)], sem_b)
            return 0

        lax.fori_loop(0, F // FQ, quad_body, 0)
        pltpu.make_async_copy(bufa_v, out_hbm.at[pl.ds(0, 2)], sem_a).wait()
        pltpu.make_async_copy(bufb_v, out_hbm.at[pl.ds(0, 2)], sem_b).wait()


def kernel(x, time_of_day, day_of_week):
    mesh = plsc.VectorSubcoreMesh(core_axis_name="c", subcore_axis_name="s",
                                  num_cores=NC, num_subcores=NS)
    out = pl.kernel(
        _body,
        out_type=jax.ShapeDtypeStruct((B * F, N), jnp.float32),
        mesh=mesh,
        compiler_params=pltpu.CompilerParams(needs_layout_passes=False),
        scratch_types=[
            pltpu.VMEM((N * C,), jnp.float32),
            pltpu.VMEM((N,), jnp.int32),
            pltpu.VMEM((TIMES * TOD_STRIDE,), jnp.float32),
            pltpu.VMEM((L * DOW_STRIDE,), jnp.float32),
            pltpu.VMEM((TIMES * TOD_STRIDE,), jnp.float32),
            pltpu.VMEM((2, N), jnp.float32),
            pltpu.VMEM((2, N), jnp.float32),
            pltpu.SemaphoreType.DMA,
            pltpu.SemaphoreType.DMA,
        ],
    )(x[:, -1].reshape(B, 1, N * C),
      time_of_day.reshape(-1), day_of_week.reshape(-1))
    return out.reshape(B, F, N, 1)


# speedup vs baseline: 1.1575x; 1.0023x over previous
import jax
import jax.numpy as jnp
from jax import lax
from jax.experimental import pallas as pl
from jax.experimental.pallas import tpu as pltpu
from jax.experimental.pallas import tpu_sc as plsc

B, T, N, C = 64, 12, 10000, 3
TIMES = 288
DAYS = 7
F = 64
NC, NS, L = 2, 16, 16
NW = NC * NS
B_PER_W = B // NW
FQ = 4
XCH = 2560
TOD_STRIDE = F + 1
DOW_STRIDE = DAYS * F + 1


def _body(x_hbm, tod_hbm, dow_hbm, out_hbm,
          slab_v, pidx_v, dowr_v, fus_v, bufa_v, bufb_v, sem_a, sem_b):
    wid = lax.axis_index("s") * NC + lax.axis_index("c")
    iota = lax.broadcasted_iota(jnp.int32, (L,), 0)

    pltpu.sync_copy(dow_hbm, slab_v.at[pl.ds(0, DAYS * F)])

    @plsc.parallel_loop(0, L, step=1)
    def _(l):
        for i in range(DAYS * F // L):
            dowr_v[pl.ds(l * DOW_STRIDE + L * i, L)] = (
                slab_v[pl.ds(L * i, L)])

    lane_off = DOW_STRIDE * iota

    for rb in range(B_PER_W):
        b = wid * B_PER_W + rb

        dminmax0 = (jnp.int32(DAYS), jnp.int32(-1))
        for tok0 in range(0, N, XCH):
            tok = min(XCH, N - tok0)
            pltpu.sync_copy(x_hbm.at[b, 0, pl.ds(3 * tok0, 3 * tok)],
                            slab_v.at[pl.ds(0, 3 * tok)])

            def idx_body_c(i, dmm, tok0=tok0):
                n0 = i * L
                base = 3 * n0 + 3 * iota
                v1 = plsc.load_gather(slab_v, [base + 1])
                v2 = plsc.load_gather(slab_v, [base + 2])
                k65 = (v1 * TIMES).astype(jnp.int32) * TOD_STRIDE
                d = v2.astype(jnp.int32)
                pidx_v[pl.ds(tok0 + n0, L)] = k65 | ((d * F) << 16)
                return (jnp.minimum(dmm[0], jnp.min(d)),
                        jnp.maximum(dmm[1], jnp.max(d)))

            dminmax0 = lax.fori_loop(0, tok // L, idx_body_c, dminmax0)

        dmin, dmax = dminmax0

        uniform_d = dmin == dmax

        pltpu.sync_copy(tod_hbm, fus_v.at[pl.ds(0, TIMES * F)])

        def restride_body(kk, _):
            k = TIMES - 1 - kk
            for i in range(F // L):
                off = (F - L) - L * i
                fus_v[pl.ds(k * TOD_STRIDE + off, L)] = (
                    fus_v[pl.ds(k * F + off, L)])
            return 0

        lax.fori_loop(0, TIMES, restride_body, 0)

        @pl.when(uniform_d)
        def _():
            d0 = dmin * F

            @plsc.parallel_loop(0, TIMES, step=1, unroll=4)
            def _(kk):
                for i in range(F // L):
                    fus_v[pl.ds(kk * TOD_STRIDE + L * i, L)] = (
                        fus_v[pl.ds(kk * TOD_STRIDE + L * i, L)]
                        + dowr_v[pl.ds(d0 + L * i, L)])

        def compute_pair(f0, buf):
            @pl.when(uniform_d)
            def _():
                @plsc.parallel_loop(0, N, step=L, unroll=5)
                def _(n0):
                    k = pidx_v[pl.ds(n0, L)] & 0xFFFF
                    for j in range(4):
                        buf[j, pl.ds(n0, L)] = (
                            plsc.load_gather(fus_v, [k + (f0 + j)]))

            @pl.when(jnp.logical_not(uniform_d))
            def _():
                @plsc.parallel_loop(0, N, step=L, unroll=5)
                def _(n0):
                    p = pidx_v[pl.ds(n0, L)]
                    k = p & 0xFFFF
                    d = lane_off + (p >> 16)
                    for j in range(4):
                        buf[j, pl.ds(n0, L)] = (
                            plsc.load_gather(fus_v, [k + (f0 + j)])
                            + plsc.load_gather(dowr_v, [d + (f0 + j)]))

        def quad_body(cc, _):
            f0 = 2 * FQ * cc
            row0 = b * F + f0

            @pl.when(cc > 0)
            def _():
                pltpu.make_async_copy(
                    bufa_v, out_hbm.at[pl.ds(0, 4)], sem_a).wait()

            compute_pair(f0, bufa_v)
            pltpu.async_copy(bufa_v, out_hbm.at[pl.ds(row0, 4)], sem_a)

            @pl.when(cc > 0)
            def _():
                pltpu.make_async_copy(
                    bufb_v, out_hbm.at[pl.ds(0, 4)], sem_b).wait()

            compute_pair(f0 + 4, bufb_v)
            pltpu.async_copy(bufb_v, out_hbm.at[pl.ds(row0 + 4, 4)], sem_b)
            return 0

        lax.fori_loop(0, F // (2 * FQ), quad_body, 0)
        pltpu.make_async_copy(bufa_v, out_hbm.at[pl.ds(0, 4)], sem_a).wait()
        pltpu.make_async_copy(bufb_v, out_hbm.at[pl.ds(0, 4)], sem_b).wait()


def kernel(x, time_of_day, day_of_week):
    mesh = plsc.VectorSubcoreMesh(core_axis_name="c", subcore_axis_name="s",
                                  num_cores=NC, num_subcores=NS)
    out = pl.kernel(
        _body,
        out_type=jax.ShapeDtypeStruct((B * F, N), jnp.float32),
        mesh=mesh,
        compiler_params=pltpu.CompilerParams(needs_layout_passes=False),
        scratch_types=[
            pltpu.VMEM((3 * XCH,), jnp.float32),
            pltpu.VMEM((N,), jnp.int32),
            pltpu.VMEM((L * DOW_STRIDE,), jnp.float32),
            pltpu.VMEM((TIMES * TOD_STRIDE,), jnp.float32),
            pltpu.VMEM((4, N), jnp.float32),
            pltpu.VMEM((4, N), jnp.float32),
            pltpu.SemaphoreType.DMA,
            pltpu.SemaphoreType.DMA,
        ],
    )(x[:, -1].reshape(B, 1, N * C),
      time_of_day.reshape(-1), day_of_week.reshape(-1))
    return out.reshape(B, F, N, 1)


# speedup vs baseline: 1.1871x; 1.0256x over previous
import jax
import jax.numpy as jnp
from jax import lax
from jax.experimental import pallas as pl
from jax.experimental.pallas import tpu as pltpu
from jax.experimental.pallas import tpu_sc as plsc

B, T, N, C = 64, 12, 10000, 3
TIMES = 288
DAYS = 7
F = 64
NC, NS, L = 2, 16, 16
NW = NC * NS
B_PER_W = B // NW
FQ = 4
XCH = 2560
TOD_STRIDE = F + 1
DOW_STRIDE = DAYS * F + 1


def _body(x_hbm, tod_hbm, dow_hbm, out_hbm,
          slab_v, pidx_v, dowr_v, fus_v, bufa_v, bufb_v, sem_a, sem_b):
    wid = lax.axis_index("s") * NC + lax.axis_index("c")
    iota = lax.broadcasted_iota(jnp.int32, (L,), 0)

    pltpu.sync_copy(dow_hbm, slab_v.at[pl.ds(0, DAYS * F)])

    @plsc.parallel_loop(0, L, step=1)
    def _(l):
        for i in range(DAYS * F // L):
            dowr_v[pl.ds(l * DOW_STRIDE + L * i, L)] = (
                slab_v[pl.ds(L * i, L)])

    lane_off = DOW_STRIDE * iota
    fus_state = jnp.int32(-2)

    for rb in range(B_PER_W):
        b = wid * B_PER_W + rb

        dminmax0 = (jnp.int32(DAYS), jnp.int32(-1))
        for tok0 in range(0, N, XCH):
            tok = min(XCH, N - tok0)
            pltpu.sync_copy(x_hbm.at[b, 0, pl.ds(3 * tok0, 3 * tok)],
                            slab_v.at[pl.ds(0, 3 * tok)])

            def idx_body_c(i, dmm, tok0=tok0):
                n0 = i * L
                base = 3 * n0 + 3 * iota
                v1 = plsc.load_gather(slab_v, [base + 1])
                v2 = plsc.load_gather(slab_v, [base + 2])
                k65 = (v1 * TIMES).astype(jnp.int32) * TOD_STRIDE
                d = v2.astype(jnp.int32)
                pidx_v[pl.ds(tok0 + n0, L)] = k65 | ((d * F) << 16)
                return (jnp.minimum(dmm[0], jnp.min(d)),
                        jnp.maximum(dmm[1], jnp.max(d)))

            dminmax0 = lax.fori_loop(0, tok // L, idx_body_c, dminmax0)

        dmin, dmax = dminmax0

        uniform_d = dmin == dmax

        want = jnp.where(uniform_d, dmin, jnp.int32(-1))
        rebuild = fus_state != want
        fus_state = want

        @pl.when(rebuild)
        def _():
            pltpu.sync_copy(tod_hbm, fus_v.at[pl.ds(0, TIMES * F)])

            def restride_body(kk, _):
                k = TIMES - 1 - kk
                for i in range(F // L):
                    off = (F - L) - L * i
                    fus_v[pl.ds(k * TOD_STRIDE + off, L)] = (
                        fus_v[pl.ds(k * F + off, L)])
                return 0

            lax.fori_loop(0, TIMES, restride_body, 0)

            @pl.when(uniform_d)
            def _():
                d0 = dmin * F

                @plsc.parallel_loop(0, TIMES, step=1, unroll=4)
                def _(kk):
                    for i in range(F // L):
                        fus_v[pl.ds(kk * TOD_STRIDE + L * i, L)] = (
                            fus_v[pl.ds(kk * TOD_STRIDE + L * i, L)]
                            + dowr_v[pl.ds(d0 + L * i, L)])

        def compute_pair(f0, buf):
            @pl.when(uniform_d)
            def _():
                @plsc.parallel_loop(0, N, step=L, unroll=5)
                def _(n0):
                    k = pidx_v[pl.ds(n0, L)] & 0xFFFF
                    for j in range(4):
                        buf[j, pl.ds(n0, L)] = (
                            plsc.load_gather(fus_v, [k + (f0 + j)]))

            @pl.when(jnp.logical_not(uniform_d))
            def _():
                @plsc.parallel_loop(0, N, step=L, unroll=5)
                def _(n0):
                    p = pidx_v[pl.ds(n0, L)]
                    k = p & 0xFFFF
                    d = lane_off + (p >> 16)
                    for j in range(4):
                        buf[j, pl.ds(n0, L)] = (
                            plsc.load_gather(fus_v, [k + (f0 + j)])
                            + plsc.load_gather(dowr_v, [d + (f0 + j)]))

        def quad_body(cc, _):
            f0 = 2 * FQ * cc
            row0 = b * F + f0

            @pl.when(cc > 0)
            def _():
                pltpu.make_async_copy(
                    bufa_v, out_hbm.at[pl.ds(0, 4)], sem_a).wait()

            compute_pair(f0, bufa_v)
            pltpu.async_copy(bufa_v, out_hbm.at[pl.ds(row0, 4)], sem_a)

            @pl.when(cc > 0)
            def _():
                pltpu.make_async_copy(
                    bufb_v, out_hbm.at[pl.ds(0, 4)], sem_b).wait()

            compute_pair(f0 + 4, bufb_v)
            pltpu.async_copy(bufb_v, out_hbm.at[pl.ds(row0 + 4, 4)], sem_b)
            return 0

        lax.fori_loop(0, F // (2 * FQ), quad_body, 0)
        pltpu.make_async_copy(bufa_v, out_hbm.at[pl.ds(0, 4)], sem_a).wait()
        pltpu.make_async_copy(bufb_v, out_hbm.at[pl.ds(0, 4)], sem_b).wait()


def kernel(x, time_of_day, day_of_week):
    mesh = plsc.VectorSubcoreMesh(core_axis_name="c", subcore_axis_name="s",
                                  num_cores=NC, num_subcores=NS)
    out = pl.kernel(
        _body,
        out_type=jax.ShapeDtypeStruct((B * F, N), jnp.float32),
        mesh=mesh,
        compiler_params=pltpu.CompilerParams(needs_layout_passes=False),
        scratch_types=[
            pltpu.VMEM((3 * XCH,), jnp.float32),
            pltpu.VMEM((N,), jnp.int32),
            pltpu.VMEM((L * DOW_STRIDE,), jnp.float32),
            pltpu.VMEM((TIMES * TOD_STRIDE,), jnp.float32),
            pltpu.VMEM((4, N), jnp.float32),
            pltpu.VMEM((4, N), jnp.float32),
            pltpu.SemaphoreType.DMA,
            pltpu.SemaphoreType.DMA,
        ],
    )(x[:, -1].reshape(B, 1, N * C),
      time_of_day.reshape(-1), day_of_week.reshape(-1))
    return out.reshape(B, F, N, 1)
